# baseline (speedup 1.0000x reference)
import functools

import jax
import jax.numpy as jnp
from jax import lax
from jax.experimental import pallas as pl
from jax.experimental.pallas import tpu as pltpu
from jax.experimental.pallas import tpu_sc as plsc

B, N, C = 16, 131072, 3
NPOINT = 16384

TILES = 32
HALVES = TILES // B
PTS = NPOINT // HALVES
WORDS = PTS * C
LANES = 16
EXPAND_STEPS = PTS // LANES


def _sc_gather(xyz_flat, point_indices):
    mesh = plsc.VectorSubcoreMesh(core_axis_name="c", subcore_axis_name="s")

    @functools.partial(
        pl.kernel,
        mesh=mesh,
        compiler_params=pltpu.CompilerParams(
            use_tc_tiling_on_sc=False, needs_layout_passes=False
        ),
        out_type=jax.ShapeDtypeStruct((B, HALVES, WORDS), jnp.float32),
        scratch_types=[
            pltpu.VMEM((PTS,), jnp.int32),
            pltpu.VMEM((WORDS,), jnp.int32),
            pltpu.VMEM((WORDS,), jnp.float32),
            pltpu.SemaphoreType.DMA,
        ],
    )
    def k(xyz_hbm, pidx_hbm, out_hbm, idx_v, idx3_v, rows_v, sem):
        wid = lax.axis_index("s") * 2 + lax.axis_index("c")
        b = wid // HALVES
        h = wid % HALVES

        pltpu.sync_copy(pidx_hbm.at[b, pl.ds(h * PTS, PTS)], idx_v)

        base = b * (N * C)
        lane3 = lax.iota(jnp.int32, LANES) * 3

        def expand(kk, carry):
            v = idx_v[pl.ds(kk * LANES, LANES)]
            v3 = v * 3 + base
            pos = lane3 + kk * (LANES * 3)
            plsc.store_scatter(idx3_v, [pos], v3)
            plsc.store_scatter(idx3_v, [pos + 1], v3 + 1)
            plsc.store_scatter(idx3_v, [pos + 2], v3 + 2)
            return carry

        lax.fori_loop(0, EXPAND_STEPS, expand, 0)

        pltpu.async_copy(xyz_hbm.at[idx3_v], rows_v, sem).wait()
        pltpu.sync_copy(rows_v, out_hbm.at[b, h])

    return k(xyz_flat, point_indices)


def kernel(xyz, point_indices):
    out = _sc_gather(xyz.reshape(B * N * C), point_indices)
    return out.reshape(B, NPOINT, C)


# speedup vs baseline: 89.3341x; 89.3341x over previous
import functools

import jax
import jax.numpy as jnp
from jax import lax
from jax.experimental import pallas as pl
from jax.experimental.pallas import tpu as pltpu
from jax.experimental.pallas import tpu_sc as plsc

B, N, C = 16, 131072, 3
NPOINT = 16384

TILES = 32
HALVES = TILES // B
PTS = NPOINT // HALVES


def _sc_gather(xyz_planar, point_indices):
    mesh = plsc.VectorSubcoreMesh(core_axis_name="c", subcore_axis_name="s")

    @functools.partial(
        pl.kernel,
        mesh=mesh,
        compiler_params=pltpu.CompilerParams(
            use_tc_tiling_on_sc=False, needs_layout_passes=False
        ),
        out_type=jax.ShapeDtypeStruct((C, B, NPOINT), jnp.float32),
        scratch_types=[
            pltpu.VMEM((PTS,), jnp.int32),
            pltpu.VMEM((PTS,), jnp.float32),
            pltpu.VMEM((PTS,), jnp.float32),
            pltpu.VMEM((PTS,), jnp.float32),
            pltpu.SemaphoreType.DMA,
        ],
    )
    def k(xyz_hbm, pidx_hbm, out_hbm, idx_v, p0_v, p1_v, p2_v, sem):
        wid = lax.axis_index("s") * 2 + lax.axis_index("c")
        b = wid // HALVES
        h = wid % HALVES

        pltpu.sync_copy(pidx_hbm.at[b, pl.ds(h * PTS, PTS)], idx_v)

        planes = (p0_v, p1_v, p2_v)
        copies = [
            pltpu.async_copy(xyz_hbm.at[c * B + b].at[idx_v], planes[c], sem)
            for c in range(C)
        ]
        for cp in copies:
            cp.wait()
        for c in range(C):
            pltpu.sync_copy(planes[c], out_hbm.at[c, b, pl.ds(h * PTS, PTS)])

    return k(xyz_planar, point_indices)


def kernel(xyz, point_indices):
    xyz_planar = jnp.transpose(xyz, (2, 0, 1)).reshape(C * B, N)
    out = _sc_gather(xyz_planar, point_indices)
    return jnp.transpose(out, (1, 2, 0))


# speedup vs baseline: 122.3730x; 1.3698x over previous
import functools

import jax
import jax.numpy as jnp
from jax import lax
from jax.experimental import pallas as pl
from jax.experimental.pallas import tpu as pltpu
from jax.experimental.pallas import tpu_sc as plsc

B, N, C = 16, 131072, 3
NPOINT = 16384

TILES = 32
HALVES = TILES // B
PTS = NPOINT // HALVES
LANES = 16
STEPS = PTS // LANES

ROW_TILE, COL_TILE = 8, 128
NT = N // COL_TILE
TILE_WORDS = ROW_TILE * COL_TILE
PLANE_ROWS = B // ROW_TILE


def _sc_gather(xyz_tiles, point_indices):
    mesh = plsc.VectorSubcoreMesh(core_axis_name="c", subcore_axis_name="s")

    @functools.partial(
        pl.kernel,
        mesh=mesh,
        compiler_params=pltpu.CompilerParams(
            use_tc_tiling_on_sc=False, needs_layout_passes=False
        ),
        out_type=jax.ShapeDtypeStruct((C, B, NPOINT), jnp.float32),
        scratch_types=[
            pltpu.VMEM((PTS,), jnp.int32),
            pltpu.VMEM((PTS,), jnp.int32),
            pltpu.VMEM((PTS,), jnp.int32),
            pltpu.VMEM((PTS,), jnp.int32),
            pltpu.VMEM((PTS,), jnp.float32),
            pltpu.VMEM((PTS,), jnp.float32),
            pltpu.VMEM((PTS,), jnp.float32),
            pltpu.SemaphoreType.DMA,
        ],
    )
    def k(xyz_hbm, pidx_hbm, out_hbm, idx_v, a0_v, a1_v, a2_v,
          p0_v, p1_v, p2_v, sem):
        wid = lax.axis_index("s") * 2 + lax.axis_index("c")
        b = wid // HALVES
        h = wid % HALVES

        pltpu.sync_copy(pidx_hbm.at[b, pl.ds(h * PTS, PTS)], idx_v)

        base = (b // ROW_TILE) * (NT * TILE_WORDS) + (b % ROW_TILE) * COL_TILE
        plane = NT * TILE_WORDS * PLANE_ROWS

        def expand(kk, carry):
            v = idx_v[pl.ds(kk * LANES, LANES)]
            t = ((v >> 7) << 10) + (v & 127) + base
            a0_v[pl.ds(kk * LANES, LANES)] = t
            a1_v[pl.ds(kk * LANES, LANES)] = t + plane
            a2_v[pl.ds(kk * LANES, LANES)] = t + 2 * plane
            return carry

        lax.fori_loop(0, STEPS, expand, 0)

        addrs = (a0_v, a1_v, a2_v)
        planes = (p0_v, p1_v, p2_v)
        copies = [
            pltpu.async_copy(xyz_hbm.at[addrs[c]], planes[c], sem)
            for c in range(C)
        ]
        for cp in copies:
            cp.wait()
        for c in range(C):
            pltpu.sync_copy(planes[c], out_hbm.at[c, b, pl.ds(h * PTS, PTS)])

    return k(xyz_tiles, point_indices)


def kernel(xyz, point_indices):
    xyz_tiles = (
        xyz.transpose(2, 0, 1)
        .reshape(C, PLANE_ROWS, ROW_TILE, NT, COL_TILE)
        .transpose(0, 1, 3, 2, 4)
        .reshape(-1)
    )
    out = _sc_gather(xyz_tiles, point_indices)
    return jnp.transpose(out, (1, 2, 0))


# speedup vs baseline: 129.7922x; 1.0606x over previous
import functools

import jax
import jax.numpy as jnp
from jax import lax
from jax.experimental import pallas as pl
from jax.experimental.pallas import tpu as pltpu
from jax.experimental.pallas import tpu_sc as plsc

B, N, C = 16, 131072, 3
NPOINT = 16384

TILES = 32
LANES = 16

ROW_TILE, COL_TILE = 8, 128
NT = N // COL_TILE
TILE_WORDS = ROW_TILE * COL_TILE
BT = B // ROW_TILE
RANGES = TILES // BT
NPT = NPOINT // COL_TILE
RPT = NPT // RANGES
PTS = RPT * ROW_TILE * COL_TILE
PLANE = BT * NT * TILE_WORDS
STEPS = PTS // LANES


def _sc_gather(xyz_tiles, pidx_tiles):
    mesh = plsc.VectorSubcoreMesh(core_axis_name="c", subcore_axis_name="s")

    @functools.partial(
        pl.kernel,
        mesh=mesh,
        compiler_params=pltpu.CompilerParams(
            use_tc_tiling_on_sc=False, needs_layout_passes=False
        ),
        out_type=jax.ShapeDtypeStruct((C, BT, RANGES, PTS), jnp.float32),
        scratch_types=[
            pltpu.VMEM((RPT, ROW_TILE, COL_TILE), jnp.int32),
            pltpu.VMEM((PTS,), jnp.int32),
            pltpu.VMEM((PTS,), jnp.int32),
            pltpu.VMEM((PTS,), jnp.int32),
            pltpu.VMEM((PTS,), jnp.float32),
            pltpu.VMEM((PTS,), jnp.float32),
            pltpu.VMEM((PTS,), jnp.float32),
            pltpu.SemaphoreType.DMA,
        ],
    )
    def k(xyz_hbm, pidx_hbm, out_hbm, idx_v, a0_v, a1_v, a2_v,
          p0_v, p1_v, p2_v, sem):
        wid = lax.axis_index("s") * 2 + lax.axis_index("c")
        bt = wid // RANGES
        r = wid % RANGES

        pltpu.sync_copy(pidx_hbm.at[bt, pl.ds(r * RPT, RPT)], idx_v)

        bt_base = bt * (NT * TILE_WORDS)

        def expand(kk, carry):
            nt_ = kk // 64
            rem = kk - nt_ * 64
            b8 = rem // 8
            i = rem - b8 * 8
            v = idx_v[nt_, b8, pl.ds(i * LANES, LANES)]
            t = ((v >> 7) << 10) + (v & 127) + (bt_base + b8 * COL_TILE)
            pos = kk * LANES
            a0_v[pl.ds(pos, LANES)] = t
            a1_v[pl.ds(pos, LANES)] = t + PLANE
            a2_v[pl.ds(pos, LANES)] = t + 2 * PLANE
            return carry

        lax.fori_loop(0, STEPS, expand, 0)

        addrs = (a0_v, a1_v, a2_v)
        planes = (p0_v, p1_v, p2_v)
        copies = [
            pltpu.async_copy(xyz_hbm.at[addrs[c]], planes[c], sem)
            for c in range(C)
        ]
        for cp in copies:
            cp.wait()
        for c in range(C):
            pltpu.sync_copy(planes[c], out_hbm.at[c, bt, r])

    return k(xyz_tiles, pidx_tiles)


def kernel(xyz, point_indices):
    xyz_tiles = (
        xyz.transpose(2, 0, 1)
        .reshape(C, BT, ROW_TILE, NT, COL_TILE)
        .transpose(0, 1, 3, 2, 4)
        .reshape(-1)
    )
    pidx_tiles = (
        point_indices.reshape(BT, ROW_TILE, NPT, COL_TILE)
        .transpose(0, 2, 1, 3)
    )
    out5 = _sc_gather(xyz_tiles, pidx_tiles)
    out = (
        out5.reshape(C, BT, NPT, ROW_TILE, COL_TILE)
        .transpose(1, 3, 2, 4, 0)
        .reshape(B, NPOINT, C)
    )
    return out


# speedup vs baseline: 135.3859x; 1.0431x over previous
import functools

import jax
import jax.numpy as jnp
from jax import lax
from jax.experimental import pallas as pl
from jax.experimental.pallas import tpu as pltpu
from jax.experimental.pallas import tpu_sc as plsc

B, N, C = 16, 131072, 3
NPOINT = 16384

TILES = 32
LANES = 16

ROW_TILE, COL_TILE = 8, 128
NT = N // COL_TILE
TILE_WORDS = ROW_TILE * COL_TILE
BT = B // ROW_TILE
RANGES = TILES // BT
NPT = NPOINT // COL_TILE
RPT = NPT // RANGES
PTS = RPT * ROW_TILE * COL_TILE
PLANE = BT * NT * TILE_WORDS
STEPS = PTS // LANES
GCHUNKS = 8


def _sc_gather(xyz_tiles, pidx_tiles):
    mesh = plsc.VectorSubcoreMesh(core_axis_name="c", subcore_axis_name="s")

    @functools.partial(
        pl.kernel,
        mesh=mesh,
        compiler_params=pltpu.CompilerParams(
            use_tc_tiling_on_sc=False, needs_layout_passes=False
        ),
        out_type=jax.ShapeDtypeStruct((C, BT, RANGES, PTS), jnp.float32),
        scratch_types=[
            pltpu.VMEM((RPT, ROW_TILE, COL_TILE), jnp.int32),
            pltpu.VMEM((PTS,), jnp.int32),
            pltpu.VMEM((PTS,), jnp.int32),
            pltpu.VMEM((PTS,), jnp.int32),
            pltpu.VMEM((PTS,), jnp.float32),
            pltpu.VMEM((PTS,), jnp.float32),
            pltpu.VMEM((PTS,), jnp.float32),
            pltpu.SemaphoreType.DMA,
        ],
    )
    def k(xyz_hbm, pidx_hbm, out_hbm, idx_v, a0_v, a1_v, a2_v,
          p0_v, p1_v, p2_v, sem):
        wid = lax.axis_index("s") * 2 + lax.axis_index("c")
        bt = wid // RANGES
        r = wid % RANGES

        pltpu.sync_copy(pidx_hbm.at[bt, pl.ds(r * RPT, RPT)], idx_v)

        bt_base = bt * (NT * TILE_WORDS)

        def expand(kk, carry):
            nt_ = kk // 64
            rem = kk - nt_ * 64
            b8 = rem // 8
            i = rem - b8 * 8
            v = idx_v[nt_, b8, pl.ds(i * LANES, LANES)]
            t = ((v >> 7) << 10) + (v & 127) + (bt_base + b8 * COL_TILE)
            pos = kk * LANES
            a0_v[pl.ds(pos, LANES)] = t
            a1_v[pl.ds(pos, LANES)] = t + PLANE
            a2_v[pl.ds(pos, LANES)] = t + 2 * PLANE
            return carry

        addrs = (a0_v, a1_v, a2_v)
        planes = (p0_v, p1_v, p2_v)
        GSTEPS = STEPS // GCHUNKS
        GPTS = PTS // GCHUNKS
        copies = []
        for g in range(GCHUNKS):
            lax.fori_loop(g * GSTEPS, (g + 1) * GSTEPS, expand, 0)
            for c in range(C):
                copies.append(
                    pltpu.async_copy(
                        xyz_hbm.at[addrs[c].at[pl.ds(g * GPTS, GPTS)]],
                        planes[c].at[pl.ds(g * GPTS, GPTS)],
                        sem,
                    )
                )
        for cp in copies:
            cp.wait()
        for c in range(C):
            pltpu.sync_copy(planes[c], out_hbm.at[c, bt, r])

    return k(xyz_tiles, pidx_tiles)


def kernel(xyz, point_indices):
    xyz_tiles = (
        xyz.transpose(2, 0, 1)
        .reshape(C, BT, ROW_TILE, NT, COL_TILE)
        .transpose(0, 1, 3, 2, 4)
        .reshape(-1)
    )
    pidx_tiles = (
        point_indices.reshape(BT, ROW_TILE, NPT, COL_TILE)
        .transpose(0, 2, 1, 3)
    )
    out5 = _sc_gather(xyz_tiles, pidx_tiles)
    out = (
        out5.reshape(C, BT, NPT, ROW_TILE, COL_TILE)
        .transpose(1, 3, 2, 4, 0)
        .reshape(B, NPOINT, C)
    )
    return out
